# baseline (speedup 1.0000x reference)
import functools

import jax
import jax.numpy as jnp
from jax import lax
from jax.experimental import pallas as pl
from jax.experimental.pallas import tpu as pltpu
from jax.experimental.pallas import tpu_sc as plsc

_NC = 2
_NS = 16
_NW = _NC * _NS
_CH = 2000
_R = 4


def _table_copy(w, src, dst, n_chunks, sems):
    n_iter = (n_chunks + _NW - 1) // _NW

    def copy(c, b):
        return pltpu.make_async_copy(
            src.at[pl.ds(c * _CH, _CH)], dst.at[pl.ds(c * _CH, _CH)], sems[b]
        )

    for j in range(n_iter):
        b = j % _R
        c = w + _NW * j

        @pl.when(c < n_chunks)
        def _():
            if j >= _R:
                copy(c - _R * _NW, b).wait()
            copy(c, b).start()

    for j in range(n_iter):
        b = j % _R
        c = w + _NW * j

        @pl.when((c < n_chunks) & (c + _R * _NW >= n_chunks))
        def _():
            copy(c, b).wait()


def _copy_kernel(u_in, i_in, u_out, i_out, s0, s1, s2, s3):
    w = lax.axis_index("s") * _NC + lax.axis_index("c")
    sems = (s0, s1, s2, s3)
    _table_copy(w, u_in, u_out, u_in.shape[0] // _CH, sems)
    _table_copy(w, i_in, i_out, i_in.shape[0] // _CH, sems)


def kernel(edge_index, user_weight, item_weight):
    mesh = plsc.VectorSubcoreMesh(core_axis_name="c", subcore_axis_name="s")
    run = functools.partial(
        pl.kernel,
        mesh=mesh,
        out_type=[
            jax.ShapeDtypeStruct(user_weight.shape, user_weight.dtype),
            jax.ShapeDtypeStruct(item_weight.shape, item_weight.dtype),
        ],
        scratch_types=[
            pltpu.SemaphoreType.DMA,
            pltpu.SemaphoreType.DMA,
            pltpu.SemaphoreType.DMA,
            pltpu.SemaphoreType.DMA,
        ],
    )(_copy_kernel)
    u_out, i_out = run(user_weight, item_weight)
    return (u_out, i_out)


# speedup vs baseline: 11.8472x; 11.8472x over previous
import functools

import jax
import jax.numpy as jnp
from jax import lax
from jax.experimental import pallas as pl
from jax.experimental.pallas import tpu as pltpu
from jax.experimental.pallas import tpu_sc as plsc

_NC = 2
_NS = 16
_NW = _NC * _NS
_CH = 1000
_D = 64


def _table_copy(w, src, dst, n_chunks, buf0, buf1, si0, si1, so0, so1):
    bufs = (buf0, buf1)
    sins = (si0, si1)
    souts = (so0, so1)
    n_iter = (n_chunks + _NW - 1) // _NW

    def in_copy(c, b):
        return pltpu.make_async_copy(src.at[pl.ds(c * _CH, _CH)], bufs[b], sins[b])

    def out_copy(c, b):
        return pltpu.make_async_copy(bufs[b], dst.at[pl.ds(c * _CH, _CH)], souts[b])

    for j in range(n_iter):
        b = j & 1
        c = w + _NW * j

        @pl.when(c < n_chunks)
        def _():
            if j >= 2:
                out_copy(c - 2 * _NW, b).wait()
            in_copy(c, b).start()
            in_copy(c, b).wait()
            out_copy(c, b).start()

    out_copy(w, 0).wait()
    out_copy(w, 1).wait()


def _copy_kernel(u_in, i_in, u_out, i_out, buf0, buf1, si0, si1, so0, so1):
    w = lax.axis_index("s") * _NC + lax.axis_index("c")
    nu = u_in.shape[0] // _CH
    ni = i_in.shape[0] // _CH
    _table_copy(w, u_in, u_out, nu, buf0, buf1, si0, si1, so0, so1)
    _table_copy(w, i_in, i_out, ni, buf0, buf1, si0, si1, so0, so1)


def kernel(edge_index, user_weight, item_weight):
    mesh = plsc.VectorSubcoreMesh(core_axis_name="c", subcore_axis_name="s")
    run = functools.partial(
        pl.kernel,
        mesh=mesh,
        out_type=[
            jax.ShapeDtypeStruct(user_weight.shape, user_weight.dtype),
            jax.ShapeDtypeStruct(item_weight.shape, item_weight.dtype),
        ],
        scratch_types=[
            pltpu.VMEM((_CH, _D), jnp.float32),
            pltpu.VMEM((_CH, _D), jnp.float32),
            pltpu.SemaphoreType.DMA,
            pltpu.SemaphoreType.DMA,
            pltpu.SemaphoreType.DMA,
            pltpu.SemaphoreType.DMA,
        ],
        compiler_params=pltpu.CompilerParams(use_tc_tiling_on_sc=False),
    )(_copy_kernel)
    u_out, i_out = run(user_weight, item_weight)
    return (u_out, i_out)


# speedup vs baseline: 15.7624x; 1.3305x over previous
import functools

import jax
import jax.numpy as jnp
from jax import lax
from jax.experimental import pallas as pl
from jax.experimental.pallas import tpu as pltpu
from jax.experimental.pallas import tpu_sc as plsc

_HBM = pltpu.MemorySpace.HBM
_NC = 2
_NS = 16
_NW = _NC * _NS
_CH = 400
_D = 64


def _sc_table_copy(w, src, dst, n_chunks, buf0, buf1, si0, si1, so0, so1):
    bufs = (buf0, buf1)
    sins = (si0, si1)
    souts = (so0, so1)
    n_iter = (n_chunks + _NW - 1) // _NW

    def in_copy(c, b):
        return pltpu.make_async_copy(src.at[pl.ds(c * _CH, _CH)], bufs[b], sins[b])

    def out_copy(c, b):
        return pltpu.make_async_copy(bufs[b], dst.at[pl.ds(c * _CH, _CH)], souts[b])

    for j in range(n_iter):
        b = j & 1
        c = w + _NW * j

        @pl.when(c < n_chunks)
        def _():
            if j >= 2:
                out_copy(c - 2 * _NW, b).wait()
            in_copy(c, b).start()
            in_copy(c, b).wait()
            out_copy(c, b).start()

    out_copy(w, 0).wait()
    out_copy(w, 1).wait()


def _sc_copy_kernel(u_in, u_out, buf0, buf1, si0, si1, so0, so1):
    w = lax.axis_index("s") * _NC + lax.axis_index("c")
    _sc_table_copy(w, u_in, u_out, u_in.shape[0] // _CH, buf0, buf1, si0, si1, so0, so1)


_TC_CH = 2000
_TC_D = 12
_TC_H = 6


def _tc_copy_body(i_in, i_out, bufs, in_sems, out_sems):
    n = i_in.shape[0] // _TC_CH

    def in_copy(c):
        b = c % _TC_D
        return pltpu.make_async_copy(
            i_in.at[pl.ds(c * _TC_CH, _TC_CH)], bufs.at[b], in_sems.at[b]
        )

    def out_copy(c):
        b = c % _TC_D
        return pltpu.make_async_copy(
            bufs.at[b], i_out.at[pl.ds(c * _TC_CH, _TC_CH)], out_sems.at[b]
        )

    for c in range(n):
        if c >= _TC_D:
            out_copy(c - _TC_D).wait()
        in_copy(c).start()
        if c >= _TC_H:
            in_copy(c - _TC_H).wait()
            out_copy(c - _TC_H).start()
    for c in range(n - _TC_H, n):
        in_copy(c).wait()
        out_copy(c).start()
    for c in range(n - _TC_D, n):
        out_copy(c).wait()


def kernel(edge_index, user_weight, item_weight):
    mesh = plsc.VectorSubcoreMesh(core_axis_name="c", subcore_axis_name="s")
    sc_run = functools.partial(
        pl.kernel,
        mesh=mesh,
        out_type=jax.ShapeDtypeStruct(user_weight.shape, user_weight.dtype),
        scratch_types=[
            pltpu.VMEM((_CH, _D), jnp.float32),
            pltpu.VMEM((_CH, _D), jnp.float32),
            pltpu.SemaphoreType.DMA,
            pltpu.SemaphoreType.DMA,
            pltpu.SemaphoreType.DMA,
            pltpu.SemaphoreType.DMA,
        ],
    )(_sc_copy_kernel)
    u_out = sc_run(user_weight)

    i_out = pl.pallas_call(
        _tc_copy_body,
        in_specs=[pl.BlockSpec(memory_space=_HBM)],
        out_specs=pl.BlockSpec(memory_space=_HBM),
        out_shape=jax.ShapeDtypeStruct(item_weight.shape, item_weight.dtype),
        scratch_shapes=[
            pltpu.VMEM((_TC_D, _TC_CH, _D), jnp.float32),
            pltpu.SemaphoreType.DMA((_TC_D,)),
            pltpu.SemaphoreType.DMA((_TC_D,)),
        ],
    )(item_weight)
    return (u_out, i_out)


# speedup vs baseline: 15.8356x; 1.0046x over previous
import functools

import jax
import jax.numpy as jnp
from jax import lax
from jax.experimental import pallas as pl
from jax.experimental.pallas import tpu as pltpu
from jax.experimental.pallas import tpu_sc as plsc

_HBM = pltpu.MemorySpace.HBM
_NC = 2
_NS = 16
_NW = _NC * _NS
_CH = 200
_R = 4
_D = 64


def _sc_copy_kernel(u_in, u_out, b0, b1, b2, b3, si0, si1, si2, si3,
                    so0, so1, so2, so3):
    w = lax.axis_index("s") * _NC + lax.axis_index("c")
    bufs = (b0, b1, b2, b3)
    sins = (si0, si1, si2, si3)
    souts = (so0, so1, so2, so3)
    n = u_in.shape[0] // _CH
    n_iter = (n + _NW - 1) // _NW

    def in_copy(c, b):
        return pltpu.make_async_copy(
            u_in.at[pl.ds(c * _CH, _CH)], bufs[b], sins[b]
        )

    def out_copy(c, b):
        return pltpu.make_async_copy(
            bufs[b], u_out.at[pl.ds(c * _CH, _CH)], souts[b]
        )

    for j in range(n_iter):
        b = j % _R
        pb = (j - 1) % _R
        c = w + _NW * j

        @pl.when(c < n)
        def _():
            if j >= _R:
                out_copy(c - _R * _NW, b).wait()
            in_copy(c, b).start()
            if j >= 1:
                in_copy(c - _NW, pb).wait()
                out_copy(c - _NW, pb).start()

    for j in range(max(0, n_iter - 2), n_iter):
        b = j % _R
        c = w + _NW * j

        @pl.when((c < n) & (c + _NW >= n))
        def _():
            in_copy(c, b).wait()
            out_copy(c, b).start()

    for j in range(max(0, n_iter - _R - 1), n_iter):
        b = j % _R
        c = w + _NW * j

        @pl.when((c < n) & (c + _R * _NW >= n))
        def _():
            out_copy(c, b).wait()


_TC_CH = 2000
_TC_D = 12
_TC_H = 6


def _tc_copy_body(i_in, i_out, bufs, in_sems, out_sems):
    n = i_in.shape[0] // _TC_CH

    def in_copy(c):
        b = c % _TC_D
        return pltpu.make_async_copy(
            i_in.at[pl.ds(c * _TC_CH, _TC_CH)], bufs.at[b], in_sems.at[b]
        )

    def out_copy(c):
        b = c % _TC_D
        return pltpu.make_async_copy(
            bufs.at[b], i_out.at[pl.ds(c * _TC_CH, _TC_CH)], out_sems.at[b]
        )

    for c in range(n):
        if c >= _TC_D:
            out_copy(c - _TC_D).wait()
        in_copy(c).start()
        if c >= _TC_H:
            in_copy(c - _TC_H).wait()
            out_copy(c - _TC_H).start()
    for c in range(n - _TC_H, n):
        in_copy(c).wait()
        out_copy(c).start()
    for c in range(n - _TC_D, n):
        out_copy(c).wait()


def kernel(edge_index, user_weight, item_weight):
    mesh = plsc.VectorSubcoreMesh(core_axis_name="c", subcore_axis_name="s")
    sc_run = functools.partial(
        pl.kernel,
        mesh=mesh,
        out_type=jax.ShapeDtypeStruct(user_weight.shape, user_weight.dtype),
        scratch_types=(
            [pltpu.VMEM((_CH, _D), jnp.float32)] * _R
            + [pltpu.SemaphoreType.DMA] * (2 * _R)
        ),
    )(_sc_copy_kernel)
    u_out = sc_run(user_weight)

    i_out = pl.pallas_call(
        _tc_copy_body,
        in_specs=[pl.BlockSpec(memory_space=_HBM)],
        out_specs=pl.BlockSpec(memory_space=_HBM),
        out_shape=jax.ShapeDtypeStruct(item_weight.shape, item_weight.dtype),
        scratch_shapes=[
            pltpu.VMEM((_TC_D, _TC_CH, _D), jnp.float32),
            pltpu.SemaphoreType.DMA((_TC_D,)),
            pltpu.SemaphoreType.DMA((_TC_D,)),
        ],
    )(item_weight)
    return (u_out, i_out)


# speedup vs baseline: 16.1553x; 1.0202x over previous
import jax
import jax.numpy as jnp
from jax.experimental import pallas as pl
from jax.experimental.pallas import tpu as pltpu

_HBM = pltpu.MemorySpace.HBM
_CH = 5000
_D = 12
_H = 6


def _copy_body(u_in, i_in, u_out, i_out, bufs, in_sems, out_sems):
    u_chunks = [(u_in, u_out, k) for k in range(u_in.shape[0] // _CH)]
    i_chunks = [(i_in, i_out, k) for k in range(i_in.shape[0] // _CH)]
    ratio = max(1, len(u_chunks) // len(i_chunks))
    chunks = []
    ui = 0
    for ic in i_chunks:
        chunks.extend(u_chunks[ui:ui + ratio])
        ui += ratio
        chunks.append(ic)
    chunks.extend(u_chunks[ui:])
    n = len(chunks)

    def in_copy(c):
        src, _, k = chunks[c]
        b = c % _D
        return pltpu.make_async_copy(
            src.at[pl.ds(k * _CH, _CH)], bufs.at[b], in_sems.at[b]
        )

    def out_copy(c):
        _, dst, k = chunks[c]
        b = c % _D
        return pltpu.make_async_copy(
            bufs.at[b], dst.at[pl.ds(k * _CH, _CH)], out_sems.at[b]
        )

    for c in range(n):
        if c >= _D:
            out_copy(c - _D).wait()
        in_copy(c).start()
        if c >= _H:
            in_copy(c - _H).wait()
            out_copy(c - _H).start()
    for c in range(n - _H, n):
        in_copy(c).wait()
        out_copy(c).start()
    for c in range(n - _D, n):
        out_copy(c).wait()


def kernel(edge_index, user_weight, item_weight):
    u_out, i_out = pl.pallas_call(
        _copy_body,
        in_specs=[
            pl.BlockSpec(memory_space=_HBM),
            pl.BlockSpec(memory_space=_HBM),
        ],
        out_specs=[
            pl.BlockSpec(memory_space=_HBM),
            pl.BlockSpec(memory_space=_HBM),
        ],
        out_shape=[
            jax.ShapeDtypeStruct(user_weight.shape, user_weight.dtype),
            jax.ShapeDtypeStruct(item_weight.shape, item_weight.dtype),
        ],
        scratch_shapes=[
            pltpu.VMEM((_D, _CH, 64), jnp.float32),
            pltpu.SemaphoreType.DMA((_D,)),
            pltpu.SemaphoreType.DMA((_D,)),
        ],
    )(user_weight, item_weight)
    return (u_out, i_out)
